# baseline (speedup 1.0000x reference)
import functools

import jax
import jax.numpy as jnp
from jax import lax
from jax.experimental import pallas as pl
from jax.experimental.pallas import tpu as pltpu

B = 1024
L = 200
V = 100000
D = 128
RPG = 16
G = B // RPG


def _pool_cos_kernel(ids_a_s, ids_b_s, mask_a_v, mask_b_v, table_hbm,
                     out_v, table_v, sem):
    @pl.when(pl.program_id(0) == 0)
    def _stage_table():
        pltpu.make_async_copy(table_hbm, table_v, sem).start()
        pltpu.make_async_copy(table_hbm, table_v, sem).wait()

    row0 = table_v[pl.ds(0, 1), :]

    def body(l, accs):
        a0, a1, b0, b1 = accs
        l1 = l + L // 2
        a0 = tuple(a0[r] + table_v[pl.ds(ids_a_s[r, l], 1), :]
                   for r in range(RPG))
        a1 = tuple(a1[r] + table_v[pl.ds(ids_a_s[r, l1], 1), :]
                   for r in range(RPG))
        b0 = tuple(b0[r] + table_v[pl.ds(ids_b_s[r, l], 1), :]
                   for r in range(RPG))
        b1 = tuple(b1[r] + table_v[pl.ds(ids_b_s[r, l1], 1), :]
                   for r in range(RPG))
        return (a0, a1, b0, b1)

    zeros = tuple(jnp.zeros((1, D), jnp.float32) for _ in range(RPG))
    a0, a1, b0, b1 = lax.fori_loop(
        0, L // 2, body, (zeros, zeros, zeros, zeros), unroll=2)

    def finish(p0, p1, mask_v):
        s = jnp.concatenate([p0[r] + p1[r] for r in range(RPG)], axis=0)
        cnt = jnp.sum(mask_v[...], axis=1, keepdims=True)
        s = s - (L - cnt) * row0
        return s / jnp.maximum(cnt, 1e-6)

    mean_a = finish(a0, a1, mask_a_v)
    mean_b = finish(b0, b1, mask_b_v)
    dot = jnp.sum(mean_a * mean_b, axis=1, keepdims=True)
    na2 = jnp.sum(mean_a * mean_a, axis=1, keepdims=True)
    nb2 = jnp.sum(mean_b * mean_b, axis=1, keepdims=True)
    cos = dot / jnp.maximum(jnp.sqrt(na2 * nb2), 1e-8) * 5.0
    out_v[...] = jnp.broadcast_to(cos, (RPG, D))


@jax.jit
def _pool_cos(ids_a_m, ids_b_m, mask_a_f, mask_b_f, feat_table):
    fn = pl.pallas_call(
        _pool_cos_kernel,
        grid=(G,),
        in_specs=[
            pl.BlockSpec((RPG, L), lambda g: (g, 0),
                         memory_space=pltpu.SMEM),
            pl.BlockSpec((RPG, L), lambda g: (g, 0),
                         memory_space=pltpu.SMEM),
            pl.BlockSpec((RPG, L), lambda g: (g, 0)),
            pl.BlockSpec((RPG, L), lambda g: (g, 0)),
            pl.BlockSpec(memory_space=pl.ANY),
        ],
        out_specs=pl.BlockSpec((RPG, D), lambda g: (g, 0)),
        out_shape=jax.ShapeDtypeStruct((B, D), jnp.float32),
        scratch_shapes=[
            pltpu.VMEM((V, D), jnp.float32),
            pltpu.SemaphoreType.DMA,
        ],
        compiler_params=pltpu.CompilerParams(
            dimension_semantics=("arbitrary",),
            vmem_limit_bytes=100 * 1024 * 1024,
        ),
    )
    out = fn(ids_a_m, ids_b_m, mask_a_f, mask_b_f, feat_table)
    return out[:, 0]


def kernel(ids_a, mask_a, ids_b, mask_b, pos_table, scale_table, rot_table,
           feat_table):
    del pos_table, scale_table, rot_table
    ids_a_m = jnp.where(mask_a, ids_a.astype(jnp.int32), 0)
    ids_b_m = jnp.where(mask_b, ids_b.astype(jnp.int32), 0)
    return _pool_cos(ids_a_m, ids_b_m, mask_a.astype(jnp.float32),
                     mask_b.astype(jnp.float32), feat_table)


# speedup vs baseline: 1.2286x; 1.2286x over previous
import functools

import jax
import jax.numpy as jnp
from jax import lax
from jax.experimental import pallas as pl
from jax.experimental.pallas import tpu as pltpu

B = 1024
L = 200
V = 100000
D = 128
RPG = 16
G = B // RPG


def _pool_cos_kernel(ids_a_s, ids_b_s, mask_a_v, mask_b_v, table_hbm,
                     out_v, table_v, sem):
    @pl.when(pl.program_id(0) == 0)
    def _stage_table():
        pltpu.make_async_copy(table_hbm, table_v, sem).start()
        pltpu.make_async_copy(table_hbm, table_v, sem).wait()

    row0 = table_v[pl.ds(0, 1), :]

    def body(l, accs):
        a0, b0 = accs
        l1 = l + L // 2
        a0 = tuple(a0[r] + (table_v[pl.ds(ids_a_s[r, l], 1), :]
                            + table_v[pl.ds(ids_a_s[r, l1], 1), :])
                   for r in range(RPG))
        b0 = tuple(b0[r] + (table_v[pl.ds(ids_b_s[r, l], 1), :]
                            + table_v[pl.ds(ids_b_s[r, l1], 1), :])
                   for r in range(RPG))
        return (a0, b0)

    zeros = tuple(jnp.zeros((1, D), jnp.float32) for _ in range(RPG))
    a0, b0 = lax.fori_loop(
        0, L // 2, body, (zeros, zeros), unroll=1)

    def finish(p0, mask_v):
        s = jnp.concatenate([p0[r] for r in range(RPG)], axis=0)
        cnt = jnp.sum(mask_v[...], axis=1, keepdims=True)
        s = s - (L - cnt) * row0
        return s / jnp.maximum(cnt, 1e-6)

    mean_a = finish(a0, mask_a_v)
    mean_b = finish(b0, mask_b_v)
    dot = jnp.sum(mean_a * mean_b, axis=1, keepdims=True)
    na2 = jnp.sum(mean_a * mean_a, axis=1, keepdims=True)
    nb2 = jnp.sum(mean_b * mean_b, axis=1, keepdims=True)
    cos = dot / jnp.maximum(jnp.sqrt(na2 * nb2), 1e-8) * 5.0
    out_v[...] = jnp.broadcast_to(cos, (RPG, D))


@jax.jit
def _pool_cos(ids_a_m, ids_b_m, mask_a_f, mask_b_f, feat_table):
    fn = pl.pallas_call(
        _pool_cos_kernel,
        grid=(G,),
        in_specs=[
            pl.BlockSpec((RPG, L), lambda g: (g, 0),
                         memory_space=pltpu.SMEM),
            pl.BlockSpec((RPG, L), lambda g: (g, 0),
                         memory_space=pltpu.SMEM),
            pl.BlockSpec((RPG, L), lambda g: (g, 0)),
            pl.BlockSpec((RPG, L), lambda g: (g, 0)),
            pl.BlockSpec(memory_space=pl.ANY),
        ],
        out_specs=pl.BlockSpec((RPG, D), lambda g: (g, 0)),
        out_shape=jax.ShapeDtypeStruct((B, D), jnp.float32),
        scratch_shapes=[
            pltpu.VMEM((V, D), jnp.float32),
            pltpu.SemaphoreType.DMA,
        ],
        compiler_params=pltpu.CompilerParams(
            dimension_semantics=("arbitrary",),
            vmem_limit_bytes=100 * 1024 * 1024,
        ),
    )
    out = fn(ids_a_m, ids_b_m, mask_a_f, mask_b_f, feat_table)
    return out[:, 0]


def kernel(ids_a, mask_a, ids_b, mask_b, pos_table, scale_table, rot_table,
           feat_table):
    del pos_table, scale_table, rot_table
    ids_a_m = jnp.where(mask_a, ids_a.astype(jnp.int32), 0)
    ids_b_m = jnp.where(mask_b, ids_b.astype(jnp.int32), 0)
    return _pool_cos(ids_a_m, ids_b_m, mask_a.astype(jnp.float32),
                     mask_b.astype(jnp.float32), feat_table)
